# baseline (speedup 1.0000x reference)
import functools

import jax
import jax.numpy as jnp
from jax import lax
from jax.experimental import pallas as pl
from jax.experimental.pallas import tpu as pltpu
from jax.experimental.pallas import tpu_sc as plsc

N_USERS = 1000000
N_ITEMS = 100000
F = 128
B = 16384

NC = 2
NS = 16
NW = NC * NS
PW = B // NW
CH = 128
NCH = PW // CH
L = 16
GPC = CH // L
JV = F // L
AP = L * (L + 1)

_mesh = plsc.VectorSubcoreMesh(core_axis_name="c", subcore_axis_name="s")


@functools.partial(
    pl.kernel,
    out_type=jax.ShapeDtypeStruct((B,), jnp.float32),
    mesh=_mesh,
    compiler_params=pltpu.CompilerParams(needs_layout_passes=False),
    scratch_types=[
        pltpu.VMEM((PW,), jnp.int32),
        pltpu.VMEM((PW,), jnp.int32),
        pltpu.VMEM((3 * CH, F), jnp.float32),
        pltpu.VMEM((3 * CH, F), jnp.float32),
        pltpu.VMEM((2 * AP,), jnp.float32),
        pltpu.VMEM((PW,), jnp.float32),
        pltpu.SemaphoreType.DMA,
        pltpu.SemaphoreType.DMA,
        pltpu.SemaphoreType.DMA,
        pltpu.SemaphoreType.DMA,
        pltpu.SemaphoreType.DMA,
        pltpu.SemaphoreType.DMA,
    ],
)
def _mf_kernel(user_hbm, item_hbm, uf_hbm, if_hbm, out_hbm,
               ui_v, ii_v, ubb, vbb, accb, outb,
               su0, su1, su2, sv0, sv1, sv2):
    wid = lax.axis_index("s") * NC + lax.axis_index("c")
    base = wid * PW

    pltpu.sync_copy(user_hbm.at[pl.ds(base, PW)], ui_v)
    pltpu.sync_copy(item_hbm.at[pl.ds(base, PW)], ii_v)

    sems = ((su0, sv0), (su1, sv1), (su2, sv2))

    def fire(c):
        p = c % 3
        su, sv = sems[p]
        hu = pltpu.async_copy(uf_hbm.at[ui_v.at[pl.ds(c * CH, CH)]],
                              ubb.at[pl.ds(p * CH, CH)], su)
        hv = pltpu.async_copy(if_hbm.at[ii_v.at[pl.ds(c * CH, CH)]],
                              vbb.at[pl.ds(p * CH, CH)], sv)
        return hu, hv

    lane = lax.iota(jnp.int32, L)

    handles = [None] * NCH
    handles[0] = fire(0)
    handles[1] = fire(1)

    def loop_body(t, _):
        for c in range(NCH):
            @pl.when(t == c * GPC)
            def _():
                hu, hv = handles[c]
                hu.wait()
                hv.wait()
                if c + 2 < NCH:
                    handles[c + 2] = fire(c + 2)

        phase = lax.rem(lax.div(t, GPC), 3)
        gbase = phase * CH + lax.rem(t, GPC) * L
        abase = lax.rem(t, 2) * AP

        RI = 8

        def row_body(r, _):
            rows = [gbase + r + k for k in range(RI)]
            accs = [ubb[rr, pl.ds(0, L)] * vbb[rr, pl.ds(0, L)]
                    for rr in rows]
            for j in range(1, JV):
                for k, rr in enumerate(rows):
                    accs[k] = accs[k] + (ubb[rr, pl.ds(j * L, L)]
                                         * vbb[rr, pl.ds(j * L, L)])
            for k in range(RI):
                accb[pl.ds(abase + (r + k) * (L + 1), L)] = accs[k]
            return 0

        lax.fori_loop(0, L // RI, lambda i, _: row_body(i * RI, _), 0,
                      unroll=True)

        rowoff = lane * (L + 1) + abase
        g0 = plsc.load_gather(accb, [rowoff])
        g1 = plsc.load_gather(accb, [rowoff + 1])
        for l in range(2, L, 2):
            g0 = g0 + plsc.load_gather(accb, [rowoff + l])
            g1 = g1 + plsc.load_gather(accb, [rowoff + l + 1])
        outb[pl.ds(t * L, L)] = g0 + g1
        return 0

    lax.fori_loop(0, NCH * GPC, loop_body, 0)

    pltpu.sync_copy(outb, out_hbm.at[pl.ds(base, PW)])


def kernel(user, item, user_factors, item_factors):
    return _mf_kernel(user.astype(jnp.int32), item.astype(jnp.int32),
                      user_factors, item_factors)


# speedup vs baseline: 1.0747x; 1.0747x over previous
import functools

import jax
import jax.numpy as jnp
from jax import lax
from jax.experimental import pallas as pl
from jax.experimental.pallas import tpu as pltpu
from jax.experimental.pallas import tpu_sc as plsc

N_USERS = 1000000
N_ITEMS = 100000
F = 128
B = 16384

NC = 2
NS = 16
NW = NC * NS
PW = B // NW
CH = 64
NCH = PW // CH
L = 16
GPC = CH // L
JV = F // L
AP = L * (L + 1)

_mesh = plsc.VectorSubcoreMesh(core_axis_name="c", subcore_axis_name="s")


@functools.partial(
    pl.kernel,
    out_type=jax.ShapeDtypeStruct((B,), jnp.float32),
    mesh=_mesh,
    compiler_params=pltpu.CompilerParams(needs_layout_passes=False),
    scratch_types=[
        pltpu.VMEM((PW,), jnp.int32),
        pltpu.VMEM((PW,), jnp.int32),
        pltpu.VMEM((3 * CH, F), jnp.float32),
        pltpu.VMEM((3 * CH, F), jnp.float32),
        pltpu.VMEM((2 * AP,), jnp.float32),
        pltpu.VMEM((PW,), jnp.float32),
        pltpu.SemaphoreType.DMA,
        pltpu.SemaphoreType.DMA,
        pltpu.SemaphoreType.DMA,
        pltpu.SemaphoreType.DMA,
        pltpu.SemaphoreType.DMA,
        pltpu.SemaphoreType.DMA,
    ],
)
def _mf_kernel(user_hbm, item_hbm, uf_hbm, if_hbm, out_hbm,
               ui_v, ii_v, ubb, vbb, accb, outb,
               su0, su1, su2, sv0, sv1, sv2):
    wid = lax.axis_index("s") * NC + lax.axis_index("c")
    base = wid * PW

    pltpu.sync_copy(user_hbm.at[pl.ds(base, PW)], ui_v)
    pltpu.sync_copy(item_hbm.at[pl.ds(base, PW)], ii_v)

    sems = ((su0, sv0), (su1, sv1), (su2, sv2))

    def fire(c):
        p = c % 3
        su, sv = sems[p]
        hu = pltpu.async_copy(uf_hbm.at[ui_v.at[pl.ds(c * CH, CH)]],
                              ubb.at[pl.ds(p * CH, CH)], su)
        hv = pltpu.async_copy(if_hbm.at[ii_v.at[pl.ds(c * CH, CH)]],
                              vbb.at[pl.ds(p * CH, CH)], sv)
        return hu, hv

    lane = lax.iota(jnp.int32, L)

    handles = [None] * NCH
    handles[0] = fire(0)
    handles[1] = fire(1)

    def loop_body(t, _):
        for c in range(NCH):
            @pl.when(t == c * GPC)
            def _():
                hu, hv = handles[c]
                hu.wait()
                hv.wait()
                if c + 2 < NCH:
                    handles[c + 2] = fire(c + 2)

        phase = lax.rem(lax.div(t, GPC), 3)
        gbase = phase * CH + lax.rem(t, GPC) * L
        abase = lax.rem(t, 2) * AP

        RI = 4

        def row_body(r, _):
            rows = [gbase + r + k for k in range(RI)]
            accs = [ubb[rr, pl.ds(0, L)] * vbb[rr, pl.ds(0, L)]
                    for rr in rows]
            for j in range(1, JV):
                for k, rr in enumerate(rows):
                    accs[k] = accs[k] + (ubb[rr, pl.ds(j * L, L)]
                                         * vbb[rr, pl.ds(j * L, L)])
            for k in range(RI):
                accb[pl.ds(abase + (r + k) * (L + 1), L)] = accs[k]
            return 0

        lax.fori_loop(0, L // RI, lambda i, _: row_body(i * RI, _), 0,
                      unroll=True)

        rowoff = lane * (L + 1) + abase
        g0 = plsc.load_gather(accb, [rowoff])
        g1 = plsc.load_gather(accb, [rowoff + 1])
        for l in range(2, L, 2):
            g0 = g0 + plsc.load_gather(accb, [rowoff + l])
            g1 = g1 + plsc.load_gather(accb, [rowoff + l + 1])
        outb[pl.ds(t * L, L)] = g0 + g1
        return 0

    lax.fori_loop(0, NCH * GPC, loop_body, 0)

    pltpu.sync_copy(outb, out_hbm.at[pl.ds(base, PW)])


def kernel(user, item, user_factors, item_factors):
    return _mf_kernel(user.astype(jnp.int32), item.astype(jnp.int32),
                      user_factors, item_factors)
